# baseline (speedup 1.0000x reference)
import functools

import jax
import jax.numpy as jnp
from jax import lax
from jax.experimental import pallas as pl
from jax.experimental.pallas import tpu as pltpu
from jax.experimental.pallas import tpu_sc as plsc

N, K, D, V, C = 10000, 32, 128, 100000, 104

NC, NS = 2, 16
NW = NC * NS
NPT = 320
N2 = NW * NPT
CH = 2
NCH = NPT // CH
E2 = N2 * K
NV = D // 16


def _sc_gather_sum(token_ids, edge_src_pad, emb):
    mesh = plsc.VectorSubcoreMesh(
        core_axis_name="c", subcore_axis_name="s", num_cores=NC, num_subcores=NS
    )
    QG = 4
    RG = CH * K
    G = NCH // QG
    assert G % 2 == 0

    @functools.partial(
        pl.kernel,
        out_type=jax.ShapeDtypeStruct((N2, D), jnp.float32),
        mesh=mesh,
        compiler_params=pltpu.CompilerParams(needs_layout_passes=False),
        scratch_types=[
            pltpu.VMEM((N,), jnp.int32),
            pltpu.VMEM((NPT * K,), jnp.int32),
            pltpu.VMEM((2, QG, RG, D), jnp.float32),
            pltpu.VMEM((2, QG * CH, D), jnp.float32),
            pltpu.SemaphoreType.DMA,
            pltpu.SemaphoreType.DMA,
            pltpu.SemaphoreType.DMA,
            pltpu.SemaphoreType.DMA,
        ],
    )
    def body(tok_hbm, edge_hbm, emb_hbm, out_hbm,
             tok_v, eidx_v, mail_v, acc_v, sem0, sem1, semo0, semo1):
        wid = lax.axis_index("s") * NC + lax.axis_index("c")
        sems = (sem0, sem1)
        semos = (semo0, semo1)
        GR = QG * CH

        with jax.named_scope("stage_in"):
            pltpu.sync_copy(tok_hbm, tok_v)
            pltpu.sync_copy(edge_hbm.at[pl.ds(wid * (NPT * K), NPT * K)],
                            eidx_v)

        def compose(t, carry):
            for q in range(8):
                e = eidx_v[pl.ds(t * 128 + q * 16, 16)]
                eidx_v[pl.ds(t * 128 + q * 16, 16)] = plsc.load_gather(
                    tok_v, [e])
            return carry
        with jax.named_scope("compose"):
            lax.fori_loop(0, NPT * K // 128, compose, 0)

        def fire(g, slot):
            for q in range(QG):
                pltpu.async_copy(
                    emb_hbm.at[eidx_v.at[pl.ds((g * QG + q) * RG, RG)]],
                    mail_v.at[slot, q], sems[slot])

        def drain(g, slot):
            for q in range(QG):
                pltpu.make_async_copy(
                    emb_hbm.at[eidx_v.at[pl.ds((g * QG + q) * RG, RG)]],
                    mail_v.at[slot, q], sems[slot]).wait()

        def reduce_group(slot):
            for q in range(QG):
                for c in range(CH):
                    acc = tuple(mail_v[slot, q, c * K, pl.ds(j * 16, 16)]
                                for j in range(NV))

                    def red(k, a, q=q, c=c):
                        return tuple(
                            a[j] + mail_v[slot, q, c * K + k,
                                          pl.ds(j * 16, 16)]
                            for j in range(NV))
                    acc = lax.fori_loop(1, K - 1, red, acc, unroll=5)
                    for j in range(NV):
                        acc_v[slot, q * CH + c, pl.ds(j * 16, 16)] = acc[j]

        def fire_out(g, slot):
            pltpu.async_copy(
                acc_v.at[slot],
                out_hbm.at[pl.ds(wid * NPT + g * GR, GR)], semos[slot])

        def drain_out(g, slot):
            pltpu.make_async_copy(
                acc_v.at[slot],
                out_hbm.at[pl.ds(wid * NPT + g * GR, GR)], semos[slot]).wait()

        with jax.named_scope("prime"):
            fire(0, 0)

        def pair(p, carry):
            g0 = 2 * p
            fire(g0 + 1, 1)
            drain(g0, 0)

            @pl.when(p > 0)
            def _():
                drain_out(g0 - 2, 0)
            reduce_group(0)
            fire_out(g0, 0)

            @pl.when(g0 + 2 < G)
            def _():
                fire(g0 + 2, 0)
            drain(g0 + 1, 1)

            @pl.when(p > 0)
            def _():
                drain_out(g0 - 1, 1)
            reduce_group(1)
            fire_out(g0 + 1, 1)
            return carry

        with jax.named_scope("mainloop"):
            lax.fori_loop(0, G // 2, pair, 0)
        with jax.named_scope("final_drain"):
            drain_out(G - 2, 0)
            drain_out(G - 1, 1)

    return body(token_ids, edge_src_pad, emb)


def _tc_dense(s, w_all, b_all, ln_g2, ln_b2, w_fc, b_fc2):
    BN = 512

    def body(s_ref, wall_ref, ball_ref, lng_ref, lnb_ref, wfc_ref, bfc_ref,
             out_ref):
        x = s_ref[...]
        g = jnp.dot(x, wall_ref[...], preferred_element_type=jnp.float32)
        g = g + ball_ref[...]
        i = jax.nn.sigmoid(g[:, :D])
        o = jax.nn.sigmoid(g[:, D:2 * D])
        u = jnp.tanh(g[:, 2 * D:])
        h = o * jnp.tanh(i * u)
        mu = jnp.mean(h, axis=-1, keepdims=True)
        var = jnp.mean(jnp.square(h - mu), axis=-1, keepdims=True)
        hn = (h - mu) / jnp.sqrt(var + 1e-5) * lng_ref[...] + lnb_ref[...]
        out_ref[...] = (
            jnp.dot(hn, wfc_ref[...], preferred_element_type=jnp.float32)
            + bfc_ref[...])

    return pl.pallas_call(
        body,
        grid=(N2 // BN,),
        in_specs=[
            pl.BlockSpec((BN, D), lambda i: (i, 0)),
            pl.BlockSpec((D, 3 * D), lambda i: (0, 0)),
            pl.BlockSpec((1, 3 * D), lambda i: (0, 0)),
            pl.BlockSpec((1, D), lambda i: (0, 0)),
            pl.BlockSpec((1, D), lambda i: (0, 0)),
            pl.BlockSpec((D, 128), lambda i: (0, 0)),
            pl.BlockSpec((1, 128), lambda i: (0, 0)),
        ],
        out_specs=pl.BlockSpec((BN, 128), lambda i: (i, 0)),
        out_shape=jax.ShapeDtypeStruct((N2, 128), jnp.float32),
    )(s, w_all, b_all, ln_g2, ln_b2, w_fc, b_fc2)


def kernel(token_ids, edge_src, emb, W_ih, b_ih, W_oh, b_oh, W_uh, b_uh,
           W_fh, b_fh, ln_g, ln_b, W_fc, b_fc):
    token_ids = token_ids.astype(jnp.int32)
    edge_src = edge_src.astype(jnp.int32)
    edge_pad = jnp.pad(edge_src, (0, E2 - N * K))

    s = _sc_gather_sum(token_ids, edge_pad, emb)

    w_all = jnp.concatenate([W_ih.T, W_oh.T, W_uh.T], axis=1)
    b_all = jnp.concatenate([b_ih, b_oh, b_uh])[None, :]
    w_fc_p = jnp.zeros((D, 128), jnp.float32).at[:, :C].set(W_fc.T)
    b_fc_p = jnp.zeros((1, 128), jnp.float32).at[0, :C].set(b_fc)

    out = _tc_dense(s, w_all, b_all, ln_g[None, :], ln_b[None, :],
                    w_fc_p, b_fc_p)
    return out[:N, :C]


# speedup vs baseline: 4.7236x; 4.7236x over previous
import functools

import jax
import jax.numpy as jnp
from jax import lax
from jax.experimental import pallas as pl
from jax.experimental.pallas import tpu as pltpu
from jax.experimental.pallas import tpu_sc as plsc

N, K, D, V, C = 10000, 32, 128, 100000, 104

NC, NS = 2, 16
NW = NC * NS
NPT = 320
N2 = NW * NPT
E2 = N2 * K
NV = D // 16

QG = 4
RG = K
GR = QG
G = NPT // GR
TB = N2 // NS
assert G % 2 == 0 and TB % (2 * RG) == 0

mesh = plsc.VectorSubcoreMesh(
    core_axis_name="c", subcore_axis_name="s", num_cores=NC, num_subcores=NS
)


def _sc_gather_sum(token_ids_pad, edge_src_pad, emb):
    @functools.partial(
        pl.kernel,
        out_type=jax.ShapeDtypeStruct((N2, D), jnp.float32),
        mesh=mesh,
        compiler_params=pltpu.CompilerParams(needs_layout_passes=False),
        scratch_types=[
            pltpu.VMEM((TB,), jnp.int32),
            pltpu.VMEM((NPT * K,), jnp.int32),
            pltpu.VMEM((2, 2, 2 * RG, D), jnp.float32),
            pltpu.VMEM((2, GR, D), jnp.float32),
            pltpu.VMEM_SHARED((N2, D), jnp.float32),
            pltpu.SemaphoreType.DMA,
            pltpu.SemaphoreType.DMA,
            pltpu.SemaphoreType.DMA,
            pltpu.SemaphoreType.DMA,
        ],
    )
    def body(tok_hbm, edge_hbm, emb_hbm, out_hbm,
             tok_v, eidx_v, mail_v, acc_v, feat_sh, sem0, sem1, semo0, semo1):
        sid = lax.axis_index("s")
        wid = sid * NC + lax.axis_index("c")
        sems = (sem0, sem1)
        semos = (semo0, semo1)

        with jax.named_scope("stage_in"):
            pltpu.sync_copy(tok_hbm.at[pl.ds(sid * TB, TB)], tok_v)
            pltpu.sync_copy(edge_hbm.at[pl.ds(wid * (NPT * K), NPT * K)],
                            eidx_v)

        P1R = 2 * RG

        def p1_src(ch):
            return emb_hbm.at[tok_v.at[pl.ds(ch * P1R, P1R)]]

        def p1_slot(slot):
            return mail_v.at[slot, 0]

        with jax.named_scope("build_feat"):
            nch = TB // P1R
            pltpu.async_copy(p1_src(0), p1_slot(0), sem0)
            for ch in range(nch):
                slot = ch % 2
                if ch + 1 < nch:
                    pltpu.async_copy(p1_src(ch + 1), p1_slot(1 - slot),
                                     sems[1 - slot])
                pltpu.make_async_copy(p1_src(ch), p1_slot(slot),
                                      sems[slot]).wait()
                pltpu.sync_copy(
                    p1_slot(slot),
                    feat_sh.at[pl.ds(sid * TB + ch * P1R, P1R)])
            plsc.subcore_barrier()

        def m_slot(slot, q):
            return mail_v.at[slot, q // 2, pl.ds(q % 2 * RG, RG)]

        def fire(g, slot):
            for q in range(QG):
                pltpu.async_copy(
                    feat_sh.at[eidx_v.at[pl.ds((g * QG + q) * RG, RG)]],
                    m_slot(slot, q), sems[slot])

        def drain(g, slot):
            for q in range(QG):
                pltpu.make_async_copy(
                    feat_sh.at[eidx_v.at[pl.ds((g * QG + q) * RG, RG)]],
                    m_slot(slot, q), sems[slot]).wait()

        def reduce_group(slot):
            for q in range(QG):
                base = q % 2 * RG
                acc = tuple(
                    mail_v[slot, q // 2, base, pl.ds(j * 16, 16)]
                    for j in range(NV))

                def red(k, a, q=q, base=base):
                    return tuple(
                        a[j] + mail_v[slot, q // 2, base + k,
                                      pl.ds(j * 16, 16)]
                        for j in range(NV))
                acc = lax.fori_loop(1, K - 1, red, acc, unroll=5)
                for j in range(NV):
                    acc_v[slot, q, pl.ds(j * 16, 16)] = acc[j]

        def fire_out(g, slot):
            pltpu.async_copy(
                acc_v.at[slot],
                out_hbm.at[pl.ds(wid * NPT + g * GR, GR)], semos[slot])

        def drain_out(g, slot):
            pltpu.make_async_copy(
                acc_v.at[slot],
                out_hbm.at[pl.ds(wid * NPT + g * GR, GR)], semos[slot]).wait()

        with jax.named_scope("prime"):
            fire(0, 0)

        def pair(p, carry):
            g0 = 2 * p
            fire(g0 + 1, 1)
            drain(g0, 0)

            @pl.when(p > 0)
            def _():
                drain_out(g0 - 2, 0)
            reduce_group(0)
            fire_out(g0, 0)

            @pl.when(g0 + 2 < G)
            def _():
                fire(g0 + 2, 0)
            drain(g0 + 1, 1)

            @pl.when(p > 0)
            def _():
                drain_out(g0 - 1, 1)
            reduce_group(1)
            fire_out(g0 + 1, 1)
            return carry

        with jax.named_scope("mainloop"):
            lax.fori_loop(0, G // 2, pair, 0)
        with jax.named_scope("final_drain"):
            drain_out(G - 2, 0)
            drain_out(G - 1, 1)

    return body(token_ids_pad, edge_src_pad, emb)


def _tc_dense(s, w_all, b_all, ln_g2, ln_b2, w_fc, b_fc2):
    BN = 512

    def body(s_ref, wall_ref, ball_ref, lng_ref, lnb_ref, wfc_ref, bfc_ref,
             out_ref):
        x = s_ref[...]
        g = jnp.dot(x, wall_ref[...], preferred_element_type=jnp.float32)
        g = g + ball_ref[...]
        i = jax.nn.sigmoid(g[:, :D])
        o = jax.nn.sigmoid(g[:, D:2 * D])
        u = jnp.tanh(g[:, 2 * D:])
        h = o * jnp.tanh(i * u)
        mu = jnp.mean(h, axis=-1, keepdims=True)
        var = jnp.mean(jnp.square(h - mu), axis=-1, keepdims=True)
        hn = (h - mu) / jnp.sqrt(var + 1e-5) * lng_ref[...] + lnb_ref[...]
        out_ref[...] = (
            jnp.dot(hn, wfc_ref[...], preferred_element_type=jnp.float32)
            + bfc_ref[...])

    return pl.pallas_call(
        body,
        grid=(N2 // BN,),
        in_specs=[
            pl.BlockSpec((BN, D), lambda i: (i, 0)),
            pl.BlockSpec((D, 3 * D), lambda i: (0, 0)),
            pl.BlockSpec((1, 3 * D), lambda i: (0, 0)),
            pl.BlockSpec((1, D), lambda i: (0, 0)),
            pl.BlockSpec((1, D), lambda i: (0, 0)),
            pl.BlockSpec((D, 128), lambda i: (0, 0)),
            pl.BlockSpec((1, 128), lambda i: (0, 0)),
        ],
        out_specs=pl.BlockSpec((BN, 128), lambda i: (i, 0)),
        out_shape=jax.ShapeDtypeStruct((N2, 128), jnp.float32),
    )(s, w_all, b_all, ln_g2, ln_b2, w_fc, b_fc2)


def kernel(token_ids, edge_src, emb, W_ih, b_ih, W_oh, b_oh, W_uh, b_uh,
           W_fh, b_fh, ln_g, ln_b, W_fc, b_fc):
    token_ids = token_ids.astype(jnp.int32)
    edge_src = edge_src.astype(jnp.int32)
    tok_pad = jnp.pad(token_ids, (0, N2 - N))
    edge_pad = jnp.pad(edge_src, (0, E2 - N * K))

    s = _sc_gather_sum(tok_pad, edge_pad, emb)

    w_all = jnp.concatenate([W_ih.T, W_oh.T, W_uh.T], axis=1)
    b_all = jnp.concatenate([b_ih, b_oh, b_uh])[None, :]
    w_fc_p = jnp.zeros((D, 128), jnp.float32).at[:, :C].set(W_fc.T)
    b_fc_p = jnp.zeros((1, 128), jnp.float32).at[0, :C].set(b_fc)

    out = _tc_dense(s, w_all, b_all, ln_g[None, :], ln_b[None, :],
                    w_fc_p, b_fc_p)
    return out[:N, :C]


# speedup vs baseline: 5.4521x; 1.1542x over previous
import functools

import jax
import jax.numpy as jnp
from jax import lax
from jax.experimental import pallas as pl
from jax.experimental.pallas import tpu as pltpu
from jax.experimental.pallas import tpu_sc as plsc

N, K, D, V, C = 10000, 32, 128, 100000, 104

NC, NS = 2, 16
NW = NC * NS
NPT = 320
N2 = NW * NPT
E2 = N2 * K
NV = D // 16

QG = 4
RG = K
GR = QG
G = NPT // GR
TB = N2 // NS
assert G % 2 == 0 and TB % (2 * RG) == 0

mesh = plsc.VectorSubcoreMesh(
    core_axis_name="c", subcore_axis_name="s", num_cores=NC, num_subcores=NS
)


def _sc_gather_sum(token_ids_pad, edge_src_pad, emb):
    @functools.partial(
        pl.kernel,
        out_type=jax.ShapeDtypeStruct((N2, D), jnp.float32),
        mesh=mesh,
        compiler_params=pltpu.CompilerParams(needs_layout_passes=False),
        scratch_types=[
            pltpu.VMEM((TB,), jnp.int32),
            pltpu.VMEM((NPT * K,), jnp.int32),
            pltpu.VMEM((2, 2, 2 * RG, D), jnp.float32),
            pltpu.VMEM((2, GR, D), jnp.float32),
            pltpu.VMEM_SHARED((N2, D), jnp.float32),
            pltpu.SemaphoreType.DMA,
            pltpu.SemaphoreType.DMA,
            pltpu.SemaphoreType.DMA,
            pltpu.SemaphoreType.DMA,
        ],
    )
    def body(tok_hbm, edge_hbm, emb_hbm, out_hbm,
             tok_v, eidx_v, mail_v, acc_v, feat_sh, sem0, sem1, semo0, semo1):
        sid = lax.axis_index("s")
        wid = sid * NC + lax.axis_index("c")
        sems = (sem0, sem1)
        semos = (semo0, semo1)

        with jax.named_scope("stage_in"):
            pltpu.sync_copy(tok_hbm.at[pl.ds(sid * TB, TB)], tok_v)
            pltpu.sync_copy(edge_hbm.at[pl.ds(wid * (NPT * K), NPT * K)],
                            eidx_v)

        P1R = 2 * RG

        def p1_src(ch):
            return emb_hbm.at[tok_v.at[pl.ds(ch * P1R, P1R)]]

        def p1_slot(slot):
            return mail_v.at[slot, 0]

        with jax.named_scope("build_feat"):
            nch = TB // P1R
            pltpu.async_copy(p1_src(0), p1_slot(0), sem0)
            for ch in range(nch):
                slot = ch % 2
                if ch + 1 < nch:
                    pltpu.async_copy(p1_src(ch + 1), p1_slot(1 - slot),
                                     sems[1 - slot])
                pltpu.make_async_copy(p1_src(ch), p1_slot(slot),
                                      sems[slot]).wait()
                pltpu.sync_copy(
                    p1_slot(slot),
                    feat_sh.at[pl.ds(sid * TB + ch * P1R, P1R)])
            plsc.subcore_barrier()

        def m_slot(slot, q):
            return mail_v.at[slot, q // 2, pl.ds(q % 2 * RG, RG)]

        def fire(g, slot):
            for q in range(QG):
                pltpu.async_copy(
                    feat_sh.at[eidx_v.at[pl.ds((g * QG + q) * RG, RG)]],
                    m_slot(slot, q), sems[slot])

        def drain(g, slot):
            for q in range(QG):
                pltpu.make_async_copy(
                    feat_sh.at[eidx_v.at[pl.ds((g * QG + q) * RG, RG)]],
                    m_slot(slot, q), sems[slot]).wait()

        def reduce_group(slot):
            for q in range(QG):
                base = q % 2 * RG
                acc = tuple(
                    mail_v[slot, q // 2, base, pl.ds(j * 16, 16)]
                    for j in range(NV))

                def red(k, a, q=q, base=base):
                    return tuple(
                        a[j] + mail_v[slot, q // 2, base + k,
                                      pl.ds(j * 16, 16)]
                        for j in range(NV))
                acc = lax.fori_loop(1, K - 1, red, acc, unroll=5)
                for j in range(NV):
                    acc_v[slot, q, pl.ds(j * 16, 16)] = acc[j]

        def fire_out(g, slot):
            pltpu.async_copy(
                acc_v.at[slot],
                out_hbm.at[pl.ds(wid * NPT + g * GR, GR)], semos[slot])

        def drain_out(g, slot):
            pltpu.make_async_copy(
                acc_v.at[slot],
                out_hbm.at[pl.ds(wid * NPT + g * GR, GR)], semos[slot]).wait()

        with jax.named_scope("prime"):
            fire(0, 0)

        def pair(p, carry):
            g0 = 2 * p
            fire(g0 + 1, 1)
            drain(g0, 0)

            @pl.when(p > 0)
            def _():
                drain_out(g0 - 2, 0)
            reduce_group(0)
            fire_out(g0, 0)

            @pl.when(g0 + 2 < G)
            def _():
                fire(g0 + 2, 0)
            drain(g0 + 1, 1)

            @pl.when(p > 0)
            def _():
                drain_out(g0 - 1, 1)
            reduce_group(1)
            fire_out(g0 + 1, 1)
            return carry

        with jax.named_scope("mainloop"):
            lax.fori_loop(0, G // 2, pair, 0)
        with jax.named_scope("final_drain"):
            drain_out(G - 2, 0)
            drain_out(G - 1, 1)

    return body(token_ids_pad, edge_src_pad, emb)


def _tc_dense(s, w_all, b_all, ln_g2, ln_b2, w_fc, b_fc2):
    BN = 512

    def body(s_ref, wall_ref, ball_ref, lng_ref, lnb_ref, wfc_ref, bfc_ref,
             out_ref):
        x = s_ref[...]
        g = jnp.dot(x, wall_ref[...], preferred_element_type=jnp.float32)
        g = g + ball_ref[...]
        i = jax.nn.sigmoid(g[:, :D])
        o = jax.nn.sigmoid(g[:, D:2 * D])
        u = jnp.tanh(g[:, 2 * D:])
        h = o * jnp.tanh(i * u)
        mu = jnp.mean(h, axis=-1, keepdims=True)
        var = jnp.mean(jnp.square(h - mu), axis=-1, keepdims=True)
        hn = (h - mu) / jnp.sqrt(var + 1e-5) * lng_ref[...] + lnb_ref[...]
        out_ref[...] = (
            jnp.dot(hn, wfc_ref[...], preferred_element_type=jnp.float32)
            + bfc_ref[...])

    return pl.pallas_call(
        body,
        grid=(N2 // BN,),
        in_specs=[
            pl.BlockSpec((BN, D), lambda i: (i, 0)),
            pl.BlockSpec((D, 3 * D), lambda i: (0, 0)),
            pl.BlockSpec((1, 3 * D), lambda i: (0, 0)),
            pl.BlockSpec((1, D), lambda i: (0, 0)),
            pl.BlockSpec((1, D), lambda i: (0, 0)),
            pl.BlockSpec((D, 128), lambda i: (0, 0)),
            pl.BlockSpec((1, 128), lambda i: (0, 0)),
        ],
        out_specs=pl.BlockSpec((BN, 128), lambda i: (i, 0)),
        out_shape=jax.ShapeDtypeStruct((N2, 128), jnp.float32),
    )(s, w_all, b_all, ln_g2, ln_b2, w_fc, b_fc2)


def kernel(token_ids, edge_src, emb, W_ih, b_ih, W_oh, b_oh, W_uh, b_uh,
           W_fh, b_fh, ln_g, ln_b, W_fc, b_fc):
    token_ids = token_ids.astype(jnp.int32)
    edge_src = edge_src.astype(jnp.int32)
    tok_pad = jnp.concatenate(
        [token_ids, jnp.arange(N2 - N, dtype=jnp.int32)])
    edge_pad = jnp.concatenate(
        [edge_src, jnp.arange(E2 - N * K, dtype=jnp.int32) % N])

    s = _sc_gather_sum(tok_pad, edge_pad, emb)

    w_all = jnp.concatenate([W_ih.T, W_oh.T, W_uh.T], axis=1)
    b_all = jnp.concatenate([b_ih, b_oh, b_uh])[None, :]
    w_fc_p = jnp.zeros((D, 128), jnp.float32).at[:, :C].set(W_fc.T)
    b_fc_p = jnp.zeros((1, 128), jnp.float32).at[0, :C].set(b_fc)

    out = _tc_dense(s, w_all, b_all, ln_g[None, :], ln_b[None, :],
                    w_fc_p, b_fc_p)
    return out[:N, :C]


# speedup vs baseline: 5.9815x; 1.0971x over previous
import functools

import jax
import jax.numpy as jnp
from jax import lax
from jax.experimental import pallas as pl
from jax.experimental.pallas import tpu as pltpu
from jax.experimental.pallas import tpu_sc as plsc

N, K, D, V, C = 10000, 32, 128, 100000, 104

NC, NS = 2, 16
NW = NC * NS
NPT = 320
N2 = NW * NPT
E2 = N2 * K
NV = D // 16

QG = 4
RG = K
GR = QG
G = NPT // GR
TB = N2 // NS
assert G % 2 == 0 and TB % (2 * RG) == 0

mesh = plsc.VectorSubcoreMesh(
    core_axis_name="c", subcore_axis_name="s", num_cores=NC, num_subcores=NS
)


def _sc_gather_sum(token_ids_pad, edge_src_pad, emb):
    @functools.partial(
        pl.kernel,
        out_type=jax.ShapeDtypeStruct((N2, D), jnp.float32),
        mesh=mesh,
        compiler_params=pltpu.CompilerParams(needs_layout_passes=False),
        scratch_types=[
            pltpu.VMEM((TB,), jnp.int32),
            pltpu.VMEM((NPT * K,), jnp.int32),
            pltpu.VMEM((2, 2, 2 * RG, D), jnp.float32),
            pltpu.VMEM((2, GR, D), jnp.float32),
            pltpu.VMEM_SHARED((N2, D), jnp.float32),
            pltpu.SemaphoreType.DMA,
            pltpu.SemaphoreType.DMA,
            pltpu.SemaphoreType.DMA,
            pltpu.SemaphoreType.DMA,
        ],
    )
    def body(tok_hbm, edge_hbm, emb_hbm, out_hbm,
             tok_v, eidx_v, mail_v, acc_v, feat_sh, sem0, sem1, semo0, semo1):
        sid = lax.axis_index("s")
        wid = sid * NC + lax.axis_index("c")
        sems = (sem0, sem1)
        semos = (semo0, semo1)

        with jax.named_scope("stage_in"):
            pltpu.sync_copy(tok_hbm.at[pl.ds(sid * TB, TB)], tok_v)
            pltpu.async_copy(edge_hbm.at[pl.ds(wid * (NPT * K), NPT * K)],
                             eidx_v, semo0)

        P1R = 2 * RG

        def p1_src(ch):
            return emb_hbm.at[tok_v.at[pl.ds(ch * P1R, P1R)]]

        def p1_slot(slot):
            return mail_v.at[slot, 0]

        with jax.named_scope("build_feat"):
            nch = TB // P1R
            pltpu.async_copy(p1_src(0), p1_slot(0), sem0)
            for ch in range(nch):
                slot = ch % 2
                if ch + 1 < nch:
                    pltpu.async_copy(p1_src(ch + 1), p1_slot(1 - slot),
                                     sems[1 - slot])
                pltpu.make_async_copy(p1_src(ch), p1_slot(slot),
                                      sems[slot]).wait()
                pltpu.sync_copy(
                    p1_slot(slot),
                    feat_sh.at[pl.ds(sid * TB + ch * P1R, P1R)])
            pltpu.make_async_copy(
                edge_hbm.at[pl.ds(wid * (NPT * K), NPT * K)],
                eidx_v, semo0).wait()
            plsc.subcore_barrier()

        def m_slot(slot, q):
            return mail_v.at[slot, q // 2, pl.ds(q % 2 * RG, RG)]

        def fire(g, slot):
            for q in range(QG):
                pltpu.async_copy(
                    feat_sh.at[eidx_v.at[pl.ds((g * QG + q) * RG, RG)]],
                    m_slot(slot, q), sems[slot])

        def drain(g, slot):
            for q in range(QG):
                pltpu.make_async_copy(
                    feat_sh.at[eidx_v.at[pl.ds((g * QG + q) * RG, RG)]],
                    m_slot(slot, q), sems[slot]).wait()

        def reduce_group(slot):
            for q in range(QG):
                base = q % 2 * RG
                acc = tuple(
                    mail_v[slot, q // 2, base, pl.ds(j * 16, 16)]
                    for j in range(NV))

                def red(k, a, q=q, base=base):
                    return tuple(
                        a[j] + mail_v[slot, q // 2, base + k,
                                      pl.ds(j * 16, 16)]
                        for j in range(NV))
                acc = lax.fori_loop(1, K - 1, red, acc, unroll=5)
                for j in range(NV):
                    acc_v[slot, q, pl.ds(j * 16, 16)] = acc[j]

        def fire_out(g, slot):
            pltpu.async_copy(
                acc_v.at[slot],
                out_hbm.at[pl.ds(wid * NPT + g * GR, GR)], semos[slot])

        def drain_out(g, slot):
            pltpu.make_async_copy(
                acc_v.at[slot],
                out_hbm.at[pl.ds(wid * NPT + g * GR, GR)], semos[slot]).wait()

        with jax.named_scope("prime"):
            fire(0, 0)

        def pair(p, carry):
            g0 = 2 * p
            fire(g0 + 1, 1)
            drain(g0, 0)

            @pl.when(p > 0)
            def _():
                drain_out(g0 - 2, 0)
            reduce_group(0)
            fire_out(g0, 0)

            @pl.when(g0 + 2 < G)
            def _():
                fire(g0 + 2, 0)
            drain(g0 + 1, 1)

            @pl.when(p > 0)
            def _():
                drain_out(g0 - 1, 1)
            reduce_group(1)
            fire_out(g0 + 1, 1)
            return carry

        with jax.named_scope("mainloop"):
            lax.fori_loop(0, G // 2, pair, 0)
        with jax.named_scope("final_drain"):
            drain_out(G - 2, 0)
            drain_out(G - 1, 1)

    return body(token_ids_pad, edge_src_pad, emb)


def _tc_dense(s, w_all, b_all, ln_g2, ln_b2, w_fc, b_fc2):
    BN = 2000

    def body(s_ref, wall_ref, ball_ref, lng_ref, lnb_ref, wfc_ref, bfc_ref,
             out_ref):
        x = s_ref[...]
        g = jnp.dot(x, wall_ref[...], preferred_element_type=jnp.float32)
        g = g + ball_ref[...]
        i = jax.nn.sigmoid(g[:, :D])
        o = jax.nn.sigmoid(g[:, D:2 * D])
        u = jnp.tanh(g[:, 2 * D:])
        h = o * jnp.tanh(i * u)
        mu = jnp.mean(h, axis=-1, keepdims=True)
        var = jnp.mean(jnp.square(h - mu), axis=-1, keepdims=True)
        hn = (h - mu) / jnp.sqrt(var + 1e-5) * lng_ref[...] + lnb_ref[...]
        out_ref[...] = (
            jnp.dot(hn, wfc_ref[...], preferred_element_type=jnp.float32)
            + bfc_ref[...])

    return pl.pallas_call(
        body,
        grid=(N // BN,),
        in_specs=[
            pl.BlockSpec((BN, D), lambda i: (i, 0)),
            pl.BlockSpec((D, 3 * D), lambda i: (0, 0)),
            pl.BlockSpec((1, 3 * D), lambda i: (0, 0)),
            pl.BlockSpec((1, D), lambda i: (0, 0)),
            pl.BlockSpec((1, D), lambda i: (0, 0)),
            pl.BlockSpec((D, C), lambda i: (0, 0)),
            pl.BlockSpec((1, C), lambda i: (0, 0)),
        ],
        out_specs=pl.BlockSpec((BN, C), lambda i: (i, 0)),
        out_shape=jax.ShapeDtypeStruct((N, C), jnp.float32),
    )(s, w_all, b_all, ln_g2, ln_b2, w_fc, b_fc2)


def kernel(token_ids, edge_src, emb, W_ih, b_ih, W_oh, b_oh, W_uh, b_uh,
           W_fh, b_fh, ln_g, ln_b, W_fc, b_fc):
    token_ids = token_ids.astype(jnp.int32)
    edge_src = edge_src.astype(jnp.int32)
    tok_pad = jnp.concatenate(
        [token_ids, jnp.arange(N2 - N, dtype=jnp.int32)])
    edge_pad = jnp.concatenate(
        [edge_src, jnp.arange(E2 - N * K, dtype=jnp.int32) % N])

    s = _sc_gather_sum(tok_pad, edge_pad, emb)

    w_all = jnp.concatenate([W_ih.T, W_oh.T, W_uh.T], axis=1)
    b_all = jnp.concatenate([b_ih, b_oh, b_uh])[None, :]

    return _tc_dense(s, w_all, b_all, ln_g[None, :], ln_b[None, :],
                     W_fc.T, b_fc[None, :])


# speedup vs baseline: 6.0474x; 1.0110x over previous
import functools

import jax
import jax.numpy as jnp
from jax import lax
from jax.experimental import pallas as pl
from jax.experimental.pallas import tpu as pltpu
from jax.experimental.pallas import tpu_sc as plsc

N, K, D, V, C = 10000, 32, 128, 100000, 104

NC, NS = 2, 16
NW = NC * NS
NPT = 320
N2 = NW * NPT
E2 = N2 * K
NV = D // 16

QG = 4
RG = K
GR = QG
G = NPT // GR
NLAST = N - (NW - 1) * NPT
G_LAST = NLAST // GR
TB = N2 // NS
assert G % 2 == 0 and G_LAST % 2 == 0 and TB % (2 * RG) == 0

mesh = plsc.VectorSubcoreMesh(
    core_axis_name="c", subcore_axis_name="s", num_cores=NC, num_subcores=NS
)


def _sc_gather_sum(token_ids_pad, edge_src_pad, emb):
    @functools.partial(
        pl.kernel,
        out_type=jax.ShapeDtypeStruct((N2, D), jnp.float32),
        mesh=mesh,
        compiler_params=pltpu.CompilerParams(needs_layout_passes=False),
        scratch_types=[
            pltpu.VMEM((TB,), jnp.int32),
            pltpu.VMEM((NPT * K,), jnp.int32),
            pltpu.VMEM((2, 2, 2 * RG, D), jnp.float32),
            pltpu.VMEM((2, GR, D), jnp.float32),
            pltpu.VMEM_SHARED((N2, D), jnp.float32),
            pltpu.SemaphoreType.DMA,
            pltpu.SemaphoreType.DMA,
            pltpu.SemaphoreType.DMA,
            pltpu.SemaphoreType.DMA,
        ],
    )
    def body(tok_hbm, edge_hbm, emb_hbm, out_hbm,
             tok_v, eidx_v, mail_v, acc_v, feat_sh, sem0, sem1, semo0, semo1):
        sid = lax.axis_index("s")
        wid = sid * NC + lax.axis_index("c")
        sems = (sem0, sem1)
        semos = (semo0, semo1)
        last = wid == NW - 1
        ng = jnp.where(last, G_LAST, G)

        with jax.named_scope("stage_in"):
            pltpu.sync_copy(tok_hbm.at[pl.ds(sid * TB, TB)], tok_v)

            @pl.when(jnp.logical_not(last))
            def _():
                pltpu.async_copy(
                    edge_hbm.at[pl.ds(wid * (NPT * K), NPT * K)],
                    eidx_v, semo0)

            @pl.when(last)
            def _():
                pltpu.async_copy(
                    edge_hbm.at[pl.ds((NW - 1) * (NPT * K), NLAST * K)],
                    eidx_v.at[pl.ds(0, NLAST * K)], semo0)

        P1R = 2 * RG

        def p1_src(ch):
            return emb_hbm.at[tok_v.at[pl.ds(ch * P1R, P1R)]]

        def p1_slot(slot):
            return mail_v.at[slot, 0]

        with jax.named_scope("build_feat"):
            nch = TB // P1R
            pltpu.async_copy(p1_src(0), p1_slot(0), sem0)
            for ch in range(nch):
                slot = ch % 2
                if ch + 1 < nch:
                    pltpu.async_copy(p1_src(ch + 1), p1_slot(1 - slot),
                                     sems[1 - slot])
                pltpu.make_async_copy(p1_src(ch), p1_slot(slot),
                                      sems[slot]).wait()
                pltpu.sync_copy(
                    p1_slot(slot),
                    feat_sh.at[pl.ds(sid * TB + ch * P1R, P1R)])
            @pl.when(jnp.logical_not(last))
            def _():
                pltpu.make_async_copy(
                    edge_hbm.at[pl.ds(wid * (NPT * K), NPT * K)],
                    eidx_v, semo0).wait()

            @pl.when(last)
            def _():
                pltpu.make_async_copy(
                    edge_hbm.at[pl.ds((NW - 1) * (NPT * K), NLAST * K)],
                    eidx_v.at[pl.ds(0, NLAST * K)], semo0).wait()
            plsc.subcore_barrier()

        def m_slot(slot, q):
            return mail_v.at[slot, q // 2, pl.ds(q % 2 * RG, RG)]

        def fire(g, slot):
            for q in range(QG):
                pltpu.async_copy(
                    feat_sh.at[eidx_v.at[pl.ds((g * QG + q) * RG, RG)]],
                    m_slot(slot, q), sems[slot])

        def drain(g, slot):
            for q in range(QG):
                pltpu.make_async_copy(
                    feat_sh.at[eidx_v.at[pl.ds((g * QG + q) * RG, RG)]],
                    m_slot(slot, q), sems[slot]).wait()

        def reduce_group(slot):
            for q in range(QG):
                base = q % 2 * RG
                acc = tuple(
                    mail_v[slot, q // 2, base, pl.ds(j * 16, 16)]
                    for j in range(NV))

                def red(k, a, q=q, base=base):
                    return tuple(
                        a[j] + mail_v[slot, q // 2, base + k,
                                      pl.ds(j * 16, 16)]
                        for j in range(NV))
                acc = lax.fori_loop(1, K - 1, red, acc, unroll=5)
                for j in range(NV):
                    acc_v[slot, q, pl.ds(j * 16, 16)] = acc[j]

        def fire_out(g, slot):
            pltpu.async_copy(
                acc_v.at[slot],
                out_hbm.at[pl.ds(wid * NPT + g * GR, GR)], semos[slot])

        def drain_out(g, slot):
            pltpu.make_async_copy(
                acc_v.at[slot],
                out_hbm.at[pl.ds(wid * NPT + g * GR, GR)], semos[slot]).wait()

        with jax.named_scope("prime"):
            fire(0, 0)

        def pair(p, carry):
            g0 = 2 * p
            fire(g0 + 1, 1)
            drain(g0, 0)

            @pl.when(p > 0)
            def _():
                drain_out(g0 - 2, 0)
            reduce_group(0)
            fire_out(g0, 0)

            @pl.when(g0 + 2 < ng)
            def _():
                fire(g0 + 2, 0)
            drain(g0 + 1, 1)

            @pl.when(p > 0)
            def _():
                drain_out(g0 - 1, 1)
            reduce_group(1)
            fire_out(g0 + 1, 1)
            return carry

        with jax.named_scope("mainloop"):
            lax.fori_loop(0, ng // 2, pair, 0)
        with jax.named_scope("final_drain"):
            drain_out(ng - 2, 0)
            drain_out(ng - 1, 1)

    return body(token_ids_pad, edge_src_pad, emb)


def _tc_dense(s, w_all, b_all, ln_g2, ln_b2, w_fc, b_fc2):
    BN = 2000

    def body(s_ref, wall_ref, ball_ref, lng_ref, lnb_ref, wfc_ref, bfc_ref,
             out_ref):
        x = s_ref[...]
        g = jnp.dot(x, wall_ref[...], preferred_element_type=jnp.float32)
        g = g + ball_ref[...]
        i = jax.nn.sigmoid(g[:, :D])
        o = jax.nn.sigmoid(g[:, D:2 * D])
        u = jnp.tanh(g[:, 2 * D:])
        h = o * jnp.tanh(i * u)
        mu = jnp.mean(h, axis=-1, keepdims=True)
        var = jnp.mean(jnp.square(h - mu), axis=-1, keepdims=True)
        hn = (h - mu) / jnp.sqrt(var + 1e-5) * lng_ref[...] + lnb_ref[...]
        out_ref[...] = (
            jnp.dot(hn, wfc_ref[...], preferred_element_type=jnp.float32)
            + bfc_ref[...])

    return pl.pallas_call(
        body,
        grid=(N // BN,),
        in_specs=[
            pl.BlockSpec((BN, D), lambda i: (i, 0)),
            pl.BlockSpec((D, 3 * D), lambda i: (0, 0)),
            pl.BlockSpec((1, 3 * D), lambda i: (0, 0)),
            pl.BlockSpec((1, D), lambda i: (0, 0)),
            pl.BlockSpec((1, D), lambda i: (0, 0)),
            pl.BlockSpec((D, C), lambda i: (0, 0)),
            pl.BlockSpec((1, C), lambda i: (0, 0)),
        ],
        out_specs=pl.BlockSpec((BN, C), lambda i: (i, 0)),
        out_shape=jax.ShapeDtypeStruct((N, C), jnp.float32),
    )(s, w_all, b_all, ln_g2, ln_b2, w_fc, b_fc2)


def kernel(token_ids, edge_src, emb, W_ih, b_ih, W_oh, b_oh, W_uh, b_uh,
           W_fh, b_fh, ln_g, ln_b, W_fc, b_fc):
    token_ids = token_ids.astype(jnp.int32)
    edge_src = edge_src.astype(jnp.int32)
    tok_pad = jnp.concatenate(
        [token_ids, jnp.arange(N2 - N, dtype=jnp.int32)])

    s = _sc_gather_sum(tok_pad, edge_src, emb)

    w_all = jnp.concatenate([W_ih.T, W_oh.T, W_uh.T], axis=1)
    b_all = jnp.concatenate([b_ih, b_oh, b_uh])[None, :]

    return _tc_dense(s, w_all, b_all, ln_g[None, :], ln_b[None, :],
                     W_fc.T, b_fc[None, :])
